# baseline (speedup 1.0000x reference)
import jax
import jax.numpy as jnp
from jax.experimental import pallas as pl
from jax.experimental.pallas import tpu as pltpu

_F32 = jnp.float32
_BF16 = jnp.bfloat16
_NBLK = 6
_NCHUNK = 2


def _net_kernel(x_ref, w3i_ref, b3i_ref, b1i_ref,
                wc_ref, bc_ref, w3_ref, b3_ref, b1_ref, wf_ref, wfo_ref,
                bf_ref, wfin_ref, bfin_ref, o_ref, feats, comp_a, comp_b,
                *, H, W):
    HW = H * W
    CH = HW // _NCHUNK
    CHE = CH + 32
    BASE = W + 16

    feats[...] = jnp.zeros(feats.shape, _BF16)
    comp_a[...] = jnp.zeros(comp_a.shape, _BF16)
    comp_b[...] = jnp.zeros(comp_b.shape, _BF16)

    wpos = jax.lax.broadcasted_iota(jnp.int32, (CH, 1), 0) % W
    ml = (wpos != 0).astype(_F32)
    mr = (wpos != (W - 1)).astype(_F32)

    def chunked(body):
        jax.lax.fori_loop(0, _NCHUNK, lambda i, _: (body(i * CH), 0)[1], 0)

    def conv_stage(rd, w3p, consume):
        def body(c0):
            x3 = jnp.concatenate(
                [rd[pl.ds(c0, CHE), :],
                 rd[pl.ds(c0 + W, CHE), :],
                 rd[pl.ds(c0 + 2 * W, CHE), :]], axis=1)
            y = jnp.dot(x3, w3p, preferred_element_type=_F32)
            o3p = (y[15:15 + CH, 0:32] * ml + y[16:16 + CH, 32:64]
                   + y[17:17 + CH, 64:96] * mr)
            consume(c0, o3p, y[16:16 + CH, 96:112])
        chunked(body)

    def compress(wr, c0, wc, bc):
        cp = jnp.maximum(
            jnp.dot(feats[pl.ds(c0, CH), :], wc,
                    preferred_element_type=_F32) + bc, 0.0)
        wr[pl.ds(BASE + c0, CH), :] = cp.astype(_BF16)

    comp_a[BASE:BASE + HW, 0:3] = x_ref[0]
    b3i = b3i_ref[...]
    b1i = b1i_ref[...]

    def init_consume(c0, o3p, o1p):
        f3 = jnp.maximum(o3p + b3i, 0.0)
        f1 = jnp.maximum(o1p + b1i, 0.0)
        feats[pl.ds(c0, CH), 0:32] = jnp.concatenate(
            [f3[:, 0:16], f1], axis=1).astype(_BF16)
        compress(comp_b, c0, wc_ref[0], bc_ref[0])
    conv_stage(comp_a, w3i_ref[...], init_consume)

    for k in range(_NBLK):
        rd, wr = (comp_b, comp_a) if k % 2 == 0 else (comp_a, comp_b)
        b3 = b3_ref[k]
        b1 = b1_ref[k]
        wf = wf_ref[k]
        wfo = wfo_ref[k]
        bf = bf_ref[k]

        def fuse_consume(c0, o3p, o1p, k=k, wr=wr,
                         b3=b3, b1=b1, wf=wf, wfo=wfo, bf=bf):
            o3 = jnp.maximum(o3p + b3, 0.0)
            o1 = jnp.maximum(o1p + b1, 0.0)
            oo = jnp.concatenate([o3, o1], axis=1).astype(_BF16)
            new = jnp.maximum(
                jnp.dot(feats[pl.ds(c0, CH), :], wf,
                        preferred_element_type=_F32)
                + jnp.dot(oo, wfo, preferred_element_type=_F32) + bf, 0.0)
            feats[pl.ds(c0, CH), 32 + 16 * k:48 + 16 * k] = new.astype(_BF16)
            if k < _NBLK - 1:
                compress(wr, c0, wc_ref[k + 1], bc_ref[k + 1])
        conv_stage(rd, w3_ref[k], fuse_consume)

    wfin = wfin_ref[...]
    bfin = bfin_ref[...]
    for c in range(_NCHUNK):
        c0 = c * CH
        out = jnp.dot(feats[c0:c0 + CH, :], wfin,
                      preferred_element_type=_F32) + bfin
        o_ref[0, :, c0:c0 + CH] = jnp.transpose(out)


def _pack_w3(w3, w1):
    cin, co3 = w3.shape[2], w3.shape[3]
    co1 = w1.shape[1]
    p = jnp.zeros((192, 112), _F32)
    for di in range(3):
        for j in range(3):
            p = p.at[64 * di:64 * di + cin, 32 * j:32 * j + co3].set(w3[di, j])
    p = p.at[64:64 + cin, 96:96 + co1].set(w1)
    return p


def kernel(x, init_w3, init_b3, init_w1, init_b1, b0_wc, b0_bc, b0_w3, b0_b3, b0_w1, b0_b1, b0_wf, b0_bf, b1_wc, b1_bc, b1_w3, b1_b3, b1_w1, b1_b1, b1_wf, b1_bf, b2_wc, b2_bc, b2_w3, b2_b3, b2_w1, b2_b1, b2_wf, b2_bf, b3_wc, b3_bc, b3_w3, b3_b3, b3_w1, b3_b1, b3_wf, b3_bf, b4_wc, b4_bc, b4_w3, b4_b3, b4_w1, b4_b1, b4_wf, b4_bf, b5_wc, b5_bc, b5_w3, b5_b3, b5_w1, b5_b1, b5_wf, b5_bf, final_w, final_b):
    N, Cin, H, W = x.shape
    HW = H * W
    blocks = [
        (b0_wc, b0_bc, b0_w3, b0_b3, b0_w1, b0_b1, b0_wf, b0_bf),
        (b1_wc, b1_bc, b1_w3, b1_b3, b1_w1, b1_b1, b1_wf, b1_bf),
        (b2_wc, b2_bc, b2_w3, b2_b3, b2_w1, b2_b1, b2_wf, b2_bf),
        (b3_wc, b3_bc, b3_w3, b3_b3, b3_w1, b3_b1, b3_wf, b3_bf),
        (b4_wc, b4_bc, b4_w3, b4_b3, b4_w1, b4_b1, b4_wf, b4_bf),
        (b5_wc, b5_bc, b5_w3, b5_b3, b5_w1, b5_b1, b5_wf, b5_bf),
    ]

    x_rows = jnp.transpose(x, (0, 2, 3, 1)).reshape(N, HW, Cin).astype(_BF16)

    w3i = _pack_w3(init_w3, init_w1).astype(_BF16)
    b3i = jnp.zeros((1, 32), _F32).at[:, 0:16].set(init_b3)

    wc_l, bc_l, w3_l, b3_l, b1_l, wf_l, wfo_l, bf_l = ([], [], [], [], [],
                                                       [], [], [])
    for k, (wc, bc, w3, b3, w1, b1, wf, bf) in enumerate(blocks):
        cin_k = 32 + 16 * k
        wc_l.append(jnp.zeros((192, 64), _F32).at[0:cin_k, :].set(wc))
        bc_l.append(bc)
        w3_l.append(_pack_w3(w3, w1))
        b3_l.append(b3)
        b1_l.append(b1)
        wf_l.append(jnp.zeros((192, 16), _F32).at[0:cin_k, :].set(wf[0:cin_k]))
        wfo_l.append(wf[cin_k:cin_k + 48])
        bf_l.append(bf)
    WC = jnp.stack(wc_l).astype(_BF16)
    BC = jnp.stack(bc_l)
    W3 = jnp.stack(w3_l).astype(_BF16)
    B3 = jnp.stack(b3_l)
    B1 = jnp.stack(b1_l)
    WF = jnp.stack(wf_l).astype(_BF16)
    WFO = jnp.stack(wfo_l).astype(_BF16)
    BF = jnp.stack(bf_l)
    wfin = jnp.zeros((192, 31), _F32).at[0:128, :].set(final_w).astype(_BF16)

    def _const(shape):
        return pl.BlockSpec(shape, lambda n: (0,) * len(shape))

    import functools
    kern = functools.partial(_net_kernel, H=H, W=W)
    out = pl.pallas_call(
        kern,
        out_shape=jax.ShapeDtypeStruct((N, 31, HW), _F32),
        grid_spec=pltpu.PrefetchScalarGridSpec(
            num_scalar_prefetch=0,
            grid=(N,),
            in_specs=[
                pl.BlockSpec((1, HW, Cin), lambda n: (n, 0, 0)),
                _const(w3i.shape), _const(b3i.shape), _const(init_b1.shape),
                _const(WC.shape), _const(BC.shape),
                _const(W3.shape), _const(B3.shape), _const(B1.shape),
                _const(WF.shape), _const(WFO.shape), _const(BF.shape),
                _const(wfin.shape), _const(final_b.shape),
            ],
            out_specs=pl.BlockSpec((1, 31, HW), lambda n: (n, 0, 0)),
            scratch_shapes=[
                pltpu.VMEM((HW, 192), _BF16),
                pltpu.VMEM((HW + 2 * W + 32, 64), _BF16),
                pltpu.VMEM((HW + 2 * W + 32, 64), _BF16),
            ],
        ),
        compiler_params=pltpu.CompilerParams(
            dimension_semantics=("parallel",)),
    )(x_rows, w3i, b3i, init_b1, WC, BC, W3, B3, B1, WF, WFO, BF,
      wfin, final_b)
    return out.reshape(N, 31, H, W)


# speedup vs baseline: 1.0757x; 1.0757x over previous
import jax
import jax.numpy as jnp
from jax.experimental import pallas as pl
from jax.experimental.pallas import tpu as pltpu

_F32 = jnp.float32
_BF16 = jnp.bfloat16
_NBLK = 6
_NCHUNK = 4


def _net_kernel(x_ref, w3i_ref, b3i_ref, b1i_ref,
                wc_ref, bc_ref, w3_ref, b3_ref, b1_ref, wf_ref, wfo_ref,
                bf_ref, wfin_ref, bfin_ref, o_ref, feats, comp_a, comp_b,
                *, H, W):
    HW = H * W
    CH = HW // _NCHUNK
    CHE = CH + 32
    BASE = W + 16

    feats[...] = jnp.zeros(feats.shape, _BF16)
    comp_a[...] = jnp.zeros(comp_a.shape, _BF16)
    comp_b[...] = jnp.zeros(comp_b.shape, _BF16)

    wpos = jax.lax.broadcasted_iota(jnp.int32, (CH, 1), 0) % W
    ml = (wpos != 0).astype(_F32)
    mr = (wpos != (W - 1)).astype(_F32)

    def chunked(body):
        jax.lax.fori_loop(0, _NCHUNK, lambda i, _: (body(i * CH), 0)[1], 0)

    def conv_stage(rd, w3p, consume):
        def body(c0):
            x3 = jnp.concatenate(
                [rd[pl.ds(c0, CHE), :],
                 rd[pl.ds(c0 + W, CHE), :],
                 rd[pl.ds(c0 + 2 * W, CHE), :]], axis=1)
            y = jnp.dot(x3, w3p, preferred_element_type=_F32)
            o3p = (y[15:15 + CH, 0:32] * ml + y[16:16 + CH, 32:64]
                   + y[17:17 + CH, 64:96] * mr)
            consume(c0, o3p, y[16:16 + CH, 96:112])
        chunked(body)

    def compress(wr, c0, wc, bc):
        cp = jnp.maximum(
            jnp.dot(feats[pl.ds(c0, CH), :], wc,
                    preferred_element_type=_F32) + bc, 0.0)
        wr[pl.ds(BASE + c0, CH), :] = cp.astype(_BF16)

    comp_a[BASE:BASE + HW, 0:3] = x_ref[0]
    b3i = b3i_ref[...]
    b1i = b1i_ref[...]

    def init_consume(c0, o3p, o1p):
        f3 = jnp.maximum(o3p + b3i, 0.0)
        f1 = jnp.maximum(o1p + b1i, 0.0)
        feats[pl.ds(c0, CH), 0:32] = jnp.concatenate(
            [f3[:, 0:16], f1], axis=1).astype(_BF16)
        compress(comp_b, c0, wc_ref[0], bc_ref[0])
    conv_stage(comp_a, w3i_ref[...], init_consume)

    for k in range(_NBLK):
        rd, wr = (comp_b, comp_a) if k % 2 == 0 else (comp_a, comp_b)
        b3 = b3_ref[k]
        b1 = b1_ref[k]
        wf = wf_ref[k]
        wfo = wfo_ref[k]
        bf = bf_ref[k]

        def fuse_consume(c0, o3p, o1p, k=k, wr=wr,
                         b3=b3, b1=b1, wf=wf, wfo=wfo, bf=bf):
            o3 = jnp.maximum(o3p + b3, 0.0)
            o1 = jnp.maximum(o1p + b1, 0.0)
            oo = jnp.concatenate([o3, o1], axis=1).astype(_BF16)
            new = jnp.maximum(
                jnp.dot(feats[pl.ds(c0, CH), :], wf,
                        preferred_element_type=_F32)
                + jnp.dot(oo, wfo, preferred_element_type=_F32) + bf, 0.0)
            feats[pl.ds(c0, CH), 32 + 16 * k:48 + 16 * k] = new.astype(_BF16)
            if k < _NBLK - 1:
                compress(wr, c0, wc_ref[k + 1], bc_ref[k + 1])
        conv_stage(rd, w3_ref[k], fuse_consume)

    wfin = wfin_ref[...]
    bfin = bfin_ref[...]
    for c in range(_NCHUNK):
        c0 = c * CH
        out = jnp.dot(feats[c0:c0 + CH, :], wfin,
                      preferred_element_type=_F32) + bfin
        o_ref[0, :, c0:c0 + CH] = jnp.transpose(out)


def _pack_w3(w3, w1):
    cin, co3 = w3.shape[2], w3.shape[3]
    co1 = w1.shape[1]
    p = jnp.zeros((192, 112), _F32)
    for di in range(3):
        for j in range(3):
            p = p.at[64 * di:64 * di + cin, 32 * j:32 * j + co3].set(w3[di, j])
    p = p.at[64:64 + cin, 96:96 + co1].set(w1)
    return p


def kernel(x, init_w3, init_b3, init_w1, init_b1, b0_wc, b0_bc, b0_w3, b0_b3, b0_w1, b0_b1, b0_wf, b0_bf, b1_wc, b1_bc, b1_w3, b1_b3, b1_w1, b1_b1, b1_wf, b1_bf, b2_wc, b2_bc, b2_w3, b2_b3, b2_w1, b2_b1, b2_wf, b2_bf, b3_wc, b3_bc, b3_w3, b3_b3, b3_w1, b3_b1, b3_wf, b3_bf, b4_wc, b4_bc, b4_w3, b4_b3, b4_w1, b4_b1, b4_wf, b4_bf, b5_wc, b5_bc, b5_w3, b5_b3, b5_w1, b5_b1, b5_wf, b5_bf, final_w, final_b):
    N, Cin, H, W = x.shape
    HW = H * W
    blocks = [
        (b0_wc, b0_bc, b0_w3, b0_b3, b0_w1, b0_b1, b0_wf, b0_bf),
        (b1_wc, b1_bc, b1_w3, b1_b3, b1_w1, b1_b1, b1_wf, b1_bf),
        (b2_wc, b2_bc, b2_w3, b2_b3, b2_w1, b2_b1, b2_wf, b2_bf),
        (b3_wc, b3_bc, b3_w3, b3_b3, b3_w1, b3_b1, b3_wf, b3_bf),
        (b4_wc, b4_bc, b4_w3, b4_b3, b4_w1, b4_b1, b4_wf, b4_bf),
        (b5_wc, b5_bc, b5_w3, b5_b3, b5_w1, b5_b1, b5_wf, b5_bf),
    ]

    x_rows = jnp.transpose(x, (0, 2, 3, 1)).reshape(N, HW, Cin).astype(_BF16)

    w3i = _pack_w3(init_w3, init_w1).astype(_BF16)
    b3i = jnp.zeros((1, 32), _F32).at[:, 0:16].set(init_b3)

    wc_l, bc_l, w3_l, b3_l, b1_l, wf_l, wfo_l, bf_l = ([], [], [], [], [],
                                                       [], [], [])
    for k, (wc, bc, w3, b3, w1, b1, wf, bf) in enumerate(blocks):
        cin_k = 32 + 16 * k
        wc_l.append(jnp.zeros((192, 64), _F32).at[0:cin_k, :].set(wc))
        bc_l.append(bc)
        w3_l.append(_pack_w3(w3, w1))
        b3_l.append(b3)
        b1_l.append(b1)
        wf_l.append(jnp.zeros((192, 16), _F32).at[0:cin_k, :].set(wf[0:cin_k]))
        wfo_l.append(wf[cin_k:cin_k + 48])
        bf_l.append(bf)
    WC = jnp.stack(wc_l).astype(_BF16)
    BC = jnp.stack(bc_l)
    W3 = jnp.stack(w3_l).astype(_BF16)
    B3 = jnp.stack(b3_l)
    B1 = jnp.stack(b1_l)
    WF = jnp.stack(wf_l).astype(_BF16)
    WFO = jnp.stack(wfo_l).astype(_BF16)
    BF = jnp.stack(bf_l)
    wfin = jnp.zeros((192, 31), _F32).at[0:128, :].set(final_w).astype(_BF16)

    def _const(shape):
        return pl.BlockSpec(shape, lambda n: (0,) * len(shape))

    import functools
    kern = functools.partial(_net_kernel, H=H, W=W)
    out = pl.pallas_call(
        kern,
        out_shape=jax.ShapeDtypeStruct((N, 31, HW), _F32),
        grid_spec=pltpu.PrefetchScalarGridSpec(
            num_scalar_prefetch=0,
            grid=(N,),
            in_specs=[
                pl.BlockSpec((1, HW, Cin), lambda n: (n, 0, 0)),
                _const(w3i.shape), _const(b3i.shape), _const(init_b1.shape),
                _const(WC.shape), _const(BC.shape),
                _const(W3.shape), _const(B3.shape), _const(B1.shape),
                _const(WF.shape), _const(WFO.shape), _const(BF.shape),
                _const(wfin.shape), _const(final_b.shape),
            ],
            out_specs=pl.BlockSpec((1, 31, HW), lambda n: (n, 0, 0)),
            scratch_shapes=[
                pltpu.VMEM((HW, 192), _BF16),
                pltpu.VMEM((HW + 2 * W + 32, 64), _BF16),
                pltpu.VMEM((HW + 2 * W + 32, 64), _BF16),
            ],
        ),
        compiler_params=pltpu.CompilerParams(
            dimension_semantics=("parallel",)),
    )(x_rows, w3i, b3i, init_b1, WC, BC, W3, B3, B1, WF, WFO, BF,
      wfin, final_b)
    return out.reshape(N, 31, H, W)
